# Initial kernel scaffold; baseline (speedup 1.0000x reference)
import jax
import jax.numpy as jnp
from jax.experimental import pallas as pl


def kernel(h, neighbors_1, neighbors_2, weights_1, weights_2, pos_1, pos_2, item, mask_item, embedding, w1_0, w2_0, w3_0, w1_1, w2_1, w3_1):
    raise NotImplementedError("write your pallas kernel here")


# speedup vs baseline: 1.2441x; 1.2441x over previous
import functools

import jax
import jax.numpy as jnp
from jax import lax
from jax.experimental import pallas as pl
from jax.experimental.pallas import tpu as pltpu
from jax.experimental.pallas import tpu_sc as plsc

B, L, S, D, P = 128, 20, 10, 128, 16
M1 = L * S
M2 = L * S * S

LP = 2 * L
N2 = B * M2
N1 = B * M1
NI = B * LP
NTOT = N2 + N1 + NI

NC, NS = 2, 16
NW = NC * NS
NPW = NTOT // NW
CH = 128
NCHUNK = NPW // CH

assert NPW * NW == NTOT and NCHUNK * CH == NPW and NCHUNK % 2 == 0


def _gather_all(idx_all, table):
    mesh = plsc.VectorSubcoreMesh(core_axis_name="c", subcore_axis_name="s")

    @functools.partial(
        pl.kernel,
        mesh=mesh,
        out_type=jax.ShapeDtypeStruct((NTOT, D), jnp.float32),
        scratch_types=[
            pltpu.VMEM((NPW,), jnp.int32),
            pltpu.VMEM((CH, D), jnp.float32),
            pltpu.VMEM((CH, D), jnp.float32),
            pltpu.SemaphoreType.DMA,
            pltpu.SemaphoreType.DMA,
            pltpu.SemaphoreType.DMA,
            pltpu.SemaphoreType.DMA,
        ],
    )
    def k(idx_hbm, tab_hbm, out_hbm, idx_v, buf0, buf1, sg0, sg1, so0, so1):
        wid = lax.axis_index("s") * NC + lax.axis_index("c")
        base = wid * NPW
        pltpu.sync_copy(idx_hbm.at[pl.ds(base, NPW)], idx_v)
        bufs = (buf0, buf1)
        sgs = (sg0, sg1)
        sos = (so0, so1)
        for b in range(2):
            pltpu.async_copy(
                tab_hbm.at[idx_v.at[pl.ds(b * CH, CH)]], bufs[b], sgs[b]
            )

        def pair(i, carry):
            for b in range(2):
                cc = 2 * i + b
                pltpu.make_async_copy(
                    tab_hbm.at[pl.ds(0, CH)], bufs[b], sgs[b]
                ).wait()
                off = pl.multiple_of(base + cc * CH, 8)
                out_cp = pltpu.async_copy(
                    bufs[b], out_hbm.at[pl.ds(off, CH)], sos[b]
                )
                out_cp.wait()

                @pl.when(cc + 2 < NCHUNK)
                def _():
                    ioff = pl.multiple_of((cc + 2) * CH, 8)
                    pltpu.async_copy(
                        tab_hbm.at[idx_v.at[pl.ds(ioff, CH)]], bufs[b], sgs[b]
                    )

            return carry

        lax.fori_loop(0, NCHUNK // 2, pair, 0)

    return k(idx_all, table)


def _agg_math(neigh, self_v, wpos, sess, w1h, w1t, w2r, w3a, w3b, m_blk):
    rows = m_blk * S
    f32 = jnp.float32
    e = neigh * sess
    t = jnp.dot(e, w1h, preferred_element_type=f32)
    t = t + jnp.dot(wpos, w1t, preferred_element_type=f32)
    t = jnp.where(t >= 0, t, 0.2 * t)
    logit = jnp.sum(t * w2r, axis=1, keepdims=True)
    p = jnp.exp(logit - jnp.max(logit))
    gid = lax.broadcasted_iota(jnp.int32, (m_blk, rows), 0)
    rid = lax.broadcasted_iota(jnp.int32, (m_blk, rows), 1) // S
    grp = (gid == rid).astype(f32)
    num = jnp.dot(grp, p * neigh, preferred_element_type=f32)
    den = jnp.dot(grp, p, preferred_element_type=f32)
    agg = num / den
    out = jnp.dot(self_v, w3a, preferred_element_type=f32)
    out = out + jnp.dot(agg, w3b, preferred_element_type=f32)
    return jnp.maximum(out, 0.0)


def _agg_a(h, gathered, mask3, wpos1, w1h, w1t, w2r, w3a, w3b):
    def body(h_ref, g1_ref, it_ref, mask_ref, wpos_ref,
             w1h_ref, w1t_ref, w2r_ref, w3a_ref, w3b_ref,
             out_ref, sess_ref):
        mask = mask_ref[0]
        items = it_ref[...]
        sess = jnp.dot(mask, items, preferred_element_type=jnp.float32)
        sess = sess / jnp.sum(mask)
        sess_ref[0] = sess
        out_ref[0] = _agg_math(
            g1_ref[...], h_ref[0], wpos_ref[0], sess,
            w1h_ref[...], w1t_ref[...], w2r_ref[...], w3a_ref[...],
            w3b_ref[...], L)

    return pl.pallas_call(
        body,
        grid=(B,),
        in_specs=[
            pl.BlockSpec((1, L, D), lambda b: (b, 0, 0)),
            pl.BlockSpec((M1, D), lambda b: (N2 // M1 + b, 0)),
            pl.BlockSpec((LP, D), lambda b: ((N2 + N1) // LP + b, 0)),
            pl.BlockSpec((1, 1, LP), lambda b: (b, 0, 0)),
            pl.BlockSpec((1, M1, 1 + P), lambda b: (b, 0, 0)),
            pl.BlockSpec((D, D), lambda b: (0, 0)),
            pl.BlockSpec((1 + P, D), lambda b: (0, 0)),
            pl.BlockSpec((1, D), lambda b: (0, 0)),
            pl.BlockSpec((D, D), lambda b: (0, 0)),
            pl.BlockSpec((D, D), lambda b: (0, 0)),
        ],
        out_specs=[
            pl.BlockSpec((1, L, D), lambda b: (b, 0, 0)),
            pl.BlockSpec((1, 1, D), lambda b: (b, 0, 0)),
        ],
        out_shape=[
            jax.ShapeDtypeStruct((B, L, D), jnp.float32),
            jax.ShapeDtypeStruct((B, 1, D), jnp.float32),
        ],
    )(h, gathered, gathered, mask3, wpos1, w1h, w1t, w2r, w3a, w3b)


MBLK_B = 40
NJ = M1 // MBLK_B


def _agg_b(gathered, wpos2, sess, w1h, w1t, w2r, w3a, w3b):
    def body(self_ref, neigh_ref, wpos_ref, sess_ref,
             w1h_ref, w1t_ref, w2r_ref, w3a_ref, w3b_ref, out_ref):
        out_ref[0] = _agg_math(
            neigh_ref[...], self_ref[...], wpos_ref[0], sess_ref[0],
            w1h_ref[...], w1t_ref[...], w2r_ref[...], w3a_ref[...],
            w3b_ref[...], MBLK_B)

    return pl.pallas_call(
        body,
        grid=(B, NJ),
        in_specs=[
            pl.BlockSpec((MBLK_B, D), lambda b, j: (N2 // MBLK_B + b * NJ + j, 0)),
            pl.BlockSpec((MBLK_B * S, D), lambda b, j: (b * NJ + j, 0)),
            pl.BlockSpec((1, MBLK_B * S, 1 + P), lambda b, j: (b, j, 0)),
            pl.BlockSpec((1, 1, D), lambda b, j: (b, 0, 0)),
            pl.BlockSpec((D, D), lambda b, j: (0, 0)),
            pl.BlockSpec((1 + P, D), lambda b, j: (0, 0)),
            pl.BlockSpec((1, D), lambda b, j: (0, 0)),
            pl.BlockSpec((D, D), lambda b, j: (0, 0)),
            pl.BlockSpec((D, D), lambda b, j: (0, 0)),
        ],
        out_specs=pl.BlockSpec((1, MBLK_B, D), lambda b, j: (b, j, 0)),
        out_shape=jax.ShapeDtypeStruct((B, M1, D), jnp.float32),
    )(gathered, gathered, wpos2, sess, w1h, w1t, w2r, w3a, w3b)


def _agg_c(out1, out2, wpos1, sess, w1h, w1t, w2r, w3a, w3b):
    def body(self_ref, neigh_ref, wpos_ref, sess_ref,
             w1h_ref, w1t_ref, w2r_ref, w3a_ref, w3b_ref, out_ref):
        out_ref[0] = _agg_math(
            neigh_ref[0], self_ref[0], wpos_ref[0], sess_ref[0],
            w1h_ref[...], w1t_ref[...], w2r_ref[...], w3a_ref[...],
            w3b_ref[...], L)

    return pl.pallas_call(
        body,
        grid=(B,),
        in_specs=[
            pl.BlockSpec((1, L, D), lambda b: (b, 0, 0)),
            pl.BlockSpec((1, M1, D), lambda b: (b, 0, 0)),
            pl.BlockSpec((1, M1, 1 + P), lambda b: (b, 0, 0)),
            pl.BlockSpec((1, 1, D), lambda b: (b, 0, 0)),
            pl.BlockSpec((D, D), lambda b: (0, 0)),
            pl.BlockSpec((1 + P, D), lambda b: (0, 0)),
            pl.BlockSpec((1, D), lambda b: (0, 0)),
            pl.BlockSpec((D, D), lambda b: (0, 0)),
            pl.BlockSpec((D, D), lambda b: (0, 0)),
        ],
        out_specs=pl.BlockSpec((1, L, D), lambda b: (b, 0, 0)),
        out_shape=jax.ShapeDtypeStruct((B, L, D), jnp.float32),
    )(out1, out2, wpos1, sess, w1h, w1t, w2r, w3a, w3b)


def kernel(h, neighbors_1, neighbors_2, weights_1, weights_2, pos_1, pos_2,
           item, mask_item, embedding, w1_0, w2_0, w3_0, w1_1, w2_1, w3_1):
    item_p = jnp.concatenate(
        [item, jnp.zeros((B, LP - L), dtype=item.dtype)], axis=1)
    idx_all = jnp.concatenate([
        neighbors_2.reshape(-1), neighbors_1.reshape(-1), item_p.reshape(-1)
    ]).astype(jnp.int32)
    gathered = _gather_all(idx_all, embedding)

    wpos1 = jnp.concatenate([weights_1[..., None], pos_1], axis=-1)
    wpos2 = jnp.concatenate([weights_2[..., None], pos_2], axis=-1)
    mask3 = jnp.concatenate(
        [mask_item, jnp.zeros((B, LP - L), dtype=mask_item.dtype)],
        axis=1).reshape(B, 1, LP)

    w1h_0, w1t_0 = w1_0[:D], w1_0[D:]
    w2r_0 = w2_0.reshape(1, D)
    w3a_0, w3b_0 = w3_0[:D], w3_0[D:]
    w1h_1, w1t_1 = w1_1[:D], w1_1[D:]
    w2r_1 = w2_1.reshape(1, D)
    w3a_1, w3b_1 = w3_1[:D], w3_1[D:]

    out1, sess = _agg_a(h, gathered, mask3, wpos1, w1h_0, w1t_0, w2r_0,
                        w3a_0, w3b_0)
    out2 = _agg_b(gathered, wpos2, sess, w1h_0, w1t_0, w2r_0, w3a_0, w3b_0)
    final = _agg_c(out1, out2, wpos1, sess, w1h_1, w1t_1, w2r_1, w3a_1, w3b_1)
    return final


# speedup vs baseline: 1.3126x; 1.0550x over previous
import functools

import jax
import jax.numpy as jnp
from jax import lax
from jax.experimental import pallas as pl
from jax.experimental.pallas import tpu as pltpu
from jax.experimental.pallas import tpu_sc as plsc

B, L, S, D, P = 128, 20, 10, 128, 16
M1 = L * S
M2 = L * S * S

LP = 2 * L
N2 = B * M2
N1 = B * M1
NI = B * LP
NTOT = N2 + N1 + NI

NC, NS = 2, 16
NW = NC * NS
NPW = NTOT // NW
CH = 128
NCHUNK = NPW // CH

assert NPW * NW == NTOT and NCHUNK * CH == NPW and NCHUNK % 2 == 0


NBUF = 5
LAG = 3

assert NCHUNK % NBUF == 0 and 0 < LAG < NBUF


def _gather_all(idx_all, table):
    mesh = plsc.VectorSubcoreMesh(core_axis_name="c", subcore_axis_name="s")

    @functools.partial(
        pl.kernel,
        mesh=mesh,
        out_type=jax.ShapeDtypeStruct((NTOT, D), jnp.float32),
        scratch_types=[
            pltpu.VMEM((NPW,), jnp.int32),
        ] + [pltpu.VMEM((CH, D), jnp.float32)] * NBUF
          + [pltpu.SemaphoreType.DMA] * (2 * NBUF),
    )
    def k(idx_hbm, tab_hbm, out_hbm, idx_v, *rest):
        bufs = rest[:NBUF]
        sgs = rest[NBUF:2 * NBUF]
        sos = rest[2 * NBUF:]
        wid = lax.axis_index("s") * NC + lax.axis_index("c")
        base = wid * NPW
        pltpu.sync_copy(idx_hbm.at[pl.ds(base, NPW)], idx_v)

        def start_gather(cc, b):
            ioff = pl.multiple_of(cc * CH, 8)
            pltpu.async_copy(
                tab_hbm.at[idx_v.at[pl.ds(ioff, CH)]], bufs[b], sgs[b])

        def wait_gather(b):
            pltpu.make_async_copy(
                tab_hbm.at[pl.ds(0, CH)], bufs[b], sgs[b]).wait()

        def start_out(cc, b):
            off = pl.multiple_of(base + cc * CH, 8)
            pltpu.async_copy(bufs[b], out_hbm.at[pl.ds(off, CH)], sos[b])

        def wait_out(b):
            pltpu.make_async_copy(
                bufs[b], out_hbm.at[pl.ds(0, CH)], sos[b]).wait()

        for c in range(LAG):
            start_gather(c, c % NBUF)

        def step(i, carry):
            for b in range(NBUF):
                c = i * NBUF + b

                bn = (b + LAG) % NBUF

                @pl.when(c + LAG < NCHUNK)
                def _():
                    @pl.when(c + LAG >= NBUF)
                    def _():
                        wait_out(bn)
                    start_gather(c + LAG, bn)

                wait_gather(b)
                start_out(c, b)
            return carry

        lax.fori_loop(0, NCHUNK // NBUF, step, 0)
        for b in range(NBUF):
            wait_out(b)

    return k(idx_all, table)


def _agg_math(neigh, self_v, wcol, pos, sess, w1h, w1w, w1p, w2r, w3a, w3b,
              m_blk):
    rows = m_blk * S
    f32 = jnp.float32
    e = neigh * sess
    t = jnp.dot(e, w1h, preferred_element_type=f32)
    t = t + jnp.dot(wcol, w1w, preferred_element_type=f32)
    t = t + jnp.dot(pos, w1p, preferred_element_type=f32)
    t = jnp.where(t >= 0, t, 0.2 * t)
    logit = jnp.sum(t * w2r, axis=1, keepdims=True)
    p = jnp.exp(logit - jnp.max(logit))
    gid = lax.broadcasted_iota(jnp.int32, (m_blk, rows), 0)
    rid = lax.broadcasted_iota(jnp.int32, (m_blk, rows), 1) // S
    grp = (gid == rid).astype(f32)
    num = jnp.dot(grp, p * neigh, preferred_element_type=f32)
    den = jnp.dot(grp, p, preferred_element_type=f32)
    agg = num / den
    out = jnp.dot(self_v, w3a, preferred_element_type=f32)
    out = out + jnp.dot(agg, w3b, preferred_element_type=f32)
    return jnp.maximum(out, 0.0)


def _agg_a(h, gathered, mask3, wcol1, pos_1, w1h, w1w, w1p, w2r, w3a, w3b):
    def body(h_ref, g1_ref, it_ref, mask_ref, wc_ref, pos_ref,
             w1h_ref, w1w_ref, w1p_ref, w2r_ref, w3a_ref, w3b_ref,
             out_ref, sess_ref):
        mask = mask_ref[0]
        items = it_ref[...]
        sess = jnp.dot(mask, items, preferred_element_type=jnp.float32)
        sess = sess / jnp.sum(mask)
        sess_ref[0] = sess
        out_ref[0] = _agg_math(
            g1_ref[...], h_ref[0], wc_ref[0], pos_ref[0], sess,
            w1h_ref[...], w1w_ref[...], w1p_ref[...], w2r_ref[...],
            w3a_ref[...], w3b_ref[...], L)

    return pl.pallas_call(
        body,
        grid=(B,),
        in_specs=[
            pl.BlockSpec((1, L, D), lambda b: (b, 0, 0)),
            pl.BlockSpec((M1, D), lambda b: (N2 // M1 + b, 0)),
            pl.BlockSpec((LP, D), lambda b: ((N2 + N1) // LP + b, 0)),
            pl.BlockSpec((1, 1, LP), lambda b: (b, 0, 0)),
            pl.BlockSpec((1, M1, 1), lambda b: (b, 0, 0)),
            pl.BlockSpec((1, M1, P), lambda b: (b, 0, 0)),
            pl.BlockSpec((D, D), lambda b: (0, 0)),
            pl.BlockSpec((1, D), lambda b: (0, 0)),
            pl.BlockSpec((P, D), lambda b: (0, 0)),
            pl.BlockSpec((1, D), lambda b: (0, 0)),
            pl.BlockSpec((D, D), lambda b: (0, 0)),
            pl.BlockSpec((D, D), lambda b: (0, 0)),
        ],
        out_specs=[
            pl.BlockSpec((1, L, D), lambda b: (b, 0, 0)),
            pl.BlockSpec((1, 1, D), lambda b: (b, 0, 0)),
        ],
        out_shape=[
            jax.ShapeDtypeStruct((B, L, D), jnp.float32),
            jax.ShapeDtypeStruct((B, 1, D), jnp.float32),
        ],
    )(h, gathered, gathered, mask3, wcol1, pos_1, w1h, w1w, w1p, w2r, w3a, w3b)


MBLK_B = 40
NJ = M1 // MBLK_B


def _agg_b(gathered, wcol2, pos_2, sess, w1h, w1w, w1p, w2r, w3a, w3b):
    def body(self_ref, neigh_ref, wc_ref, pos_ref, sess_ref,
             w1h_ref, w1w_ref, w1p_ref, w2r_ref, w3a_ref, w3b_ref, out_ref):
        out_ref[0] = _agg_math(
            neigh_ref[...], self_ref[...], wc_ref[0], pos_ref[0], sess_ref[0],
            w1h_ref[...], w1w_ref[...], w1p_ref[...], w2r_ref[...],
            w3a_ref[...], w3b_ref[...], MBLK_B)

    return pl.pallas_call(
        body,
        grid=(B, NJ),
        in_specs=[
            pl.BlockSpec((MBLK_B, D), lambda b, j: (N2 // MBLK_B + b * NJ + j, 0)),
            pl.BlockSpec((MBLK_B * S, D), lambda b, j: (b * NJ + j, 0)),
            pl.BlockSpec((1, MBLK_B * S, 1), lambda b, j: (b, j, 0)),
            pl.BlockSpec((1, MBLK_B * S, P), lambda b, j: (b, j, 0)),
            pl.BlockSpec((1, 1, D), lambda b, j: (b, 0, 0)),
            pl.BlockSpec((D, D), lambda b, j: (0, 0)),
            pl.BlockSpec((1, D), lambda b, j: (0, 0)),
            pl.BlockSpec((P, D), lambda b, j: (0, 0)),
            pl.BlockSpec((1, D), lambda b, j: (0, 0)),
            pl.BlockSpec((D, D), lambda b, j: (0, 0)),
            pl.BlockSpec((D, D), lambda b, j: (0, 0)),
        ],
        out_specs=pl.BlockSpec((1, MBLK_B, D), lambda b, j: (b, j, 0)),
        out_shape=jax.ShapeDtypeStruct((B, M1, D), jnp.float32),
    )(gathered, gathered, wcol2, pos_2, sess, w1h, w1w, w1p, w2r, w3a, w3b)


def _agg_c(out1, out2, wcol1, pos_1, sess, w1h, w1w, w1p, w2r, w3a, w3b):
    def body(self_ref, neigh_ref, wc_ref, pos_ref, sess_ref,
             w1h_ref, w1w_ref, w1p_ref, w2r_ref, w3a_ref, w3b_ref, out_ref):
        out_ref[0] = _agg_math(
            neigh_ref[0], self_ref[0], wc_ref[0], pos_ref[0], sess_ref[0],
            w1h_ref[...], w1w_ref[...], w1p_ref[...], w2r_ref[...],
            w3a_ref[...], w3b_ref[...], L)

    return pl.pallas_call(
        body,
        grid=(B,),
        in_specs=[
            pl.BlockSpec((1, L, D), lambda b: (b, 0, 0)),
            pl.BlockSpec((1, M1, D), lambda b: (b, 0, 0)),
            pl.BlockSpec((1, M1, 1), lambda b: (b, 0, 0)),
            pl.BlockSpec((1, M1, P), lambda b: (b, 0, 0)),
            pl.BlockSpec((1, 1, D), lambda b: (b, 0, 0)),
            pl.BlockSpec((D, D), lambda b: (0, 0)),
            pl.BlockSpec((1, D), lambda b: (0, 0)),
            pl.BlockSpec((P, D), lambda b: (0, 0)),
            pl.BlockSpec((1, D), lambda b: (0, 0)),
            pl.BlockSpec((D, D), lambda b: (0, 0)),
            pl.BlockSpec((D, D), lambda b: (0, 0)),
        ],
        out_specs=pl.BlockSpec((1, L, D), lambda b: (b, 0, 0)),
        out_shape=jax.ShapeDtypeStruct((B, L, D), jnp.float32),
    )(out1, out2, wcol1, pos_1, sess, w1h, w1w, w1p, w2r, w3a, w3b)


def kernel(h, neighbors_1, neighbors_2, weights_1, weights_2, pos_1, pos_2,
           item, mask_item, embedding, w1_0, w2_0, w3_0, w1_1, w2_1, w3_1):
    item_p = jnp.concatenate(
        [item, jnp.zeros((B, LP - L), dtype=item.dtype)], axis=1)
    idx_all = jnp.concatenate([
        neighbors_2.reshape(-1), neighbors_1.reshape(-1), item_p.reshape(-1)
    ]).astype(jnp.int32)
    gathered = _gather_all(idx_all, embedding)

    wcol1 = weights_1.reshape(B, M1, 1)
    wcol2 = weights_2.reshape(B, M2, 1)
    mask3 = jnp.concatenate(
        [mask_item, jnp.zeros((B, LP - L), dtype=mask_item.dtype)],
        axis=1).reshape(B, 1, LP)

    w1h_0, w1w_0, w1p_0 = w1_0[:D], w1_0[D:D + 1], w1_0[D + 1:]
    w2r_0 = w2_0.reshape(1, D)
    w3a_0, w3b_0 = w3_0[:D], w3_0[D:]
    w1h_1, w1w_1, w1p_1 = w1_1[:D], w1_1[D:D + 1], w1_1[D + 1:]
    w2r_1 = w2_1.reshape(1, D)
    w3a_1, w3b_1 = w3_1[:D], w3_1[D:]

    out1, sess = _agg_a(h, gathered, mask3, wcol1, pos_1,
                        w1h_0, w1w_0, w1p_0, w2r_0, w3a_0, w3b_0)
    out2 = _agg_b(gathered, wcol2, pos_2, sess,
                  w1h_0, w1w_0, w1p_0, w2r_0, w3a_0, w3b_0)
    final = _agg_c(out1, out2, wcol1, pos_1, sess,
                   w1h_1, w1w_1, w1p_1, w2r_1, w3a_1, w3b_1)
    return final


# speedup vs baseline: 1.3517x; 1.0298x over previous
import functools

import jax
import jax.numpy as jnp
from jax import lax
from jax.experimental import pallas as pl
from jax.experimental.pallas import tpu as pltpu
from jax.experimental.pallas import tpu_sc as plsc

B, L, S, D, P = 128, 20, 10, 128, 16
M1 = L * S
M2 = L * S * S

LP = 2 * L
N1 = B * M1
NI = B * LP
N2 = B * M2
NTOT = N1 + NI + N2

KPART = 4
BQ = B // KPART
NPART = BQ * M2

NC, NS = 2, 16
NW = NC * NS


def _gather_region(idx_all, table, start, nrows, ch, nbuf, lag):
    npw = nrows // NW
    nchunk = npw // ch
    assert npw * NW == nrows and nchunk * ch == npw and nchunk % nbuf == 0
    assert ch <= 128 and ch % 8 == 0 and 0 < lag < nbuf
    mesh = plsc.VectorSubcoreMesh(core_axis_name="c", subcore_axis_name="s")

    @functools.partial(
        pl.kernel,
        mesh=mesh,
        out_type=jax.ShapeDtypeStruct((nrows, D), jnp.float32),
        scratch_types=[
            pltpu.VMEM((npw,), jnp.int32),
        ] + [pltpu.VMEM((ch, D), jnp.float32)] * nbuf
          + [pltpu.SemaphoreType.DMA] * (2 * nbuf),
    )
    def k(idx_hbm, tab_hbm, out_hbm, idx_v, *rest):
        bufs = rest[:nbuf]
        sgs = rest[nbuf:2 * nbuf]
        sos = rest[2 * nbuf:]
        wid = lax.axis_index("s") * NC + lax.axis_index("c")
        base = wid * npw
        pltpu.sync_copy(idx_hbm.at[pl.ds(start + base, npw)], idx_v)

        def start_gather(cc, b):
            ioff = pl.multiple_of(cc * ch, 8)
            pltpu.async_copy(
                tab_hbm.at[idx_v.at[pl.ds(ioff, ch)]], bufs[b], sgs[b])

        def wait_gather(b):
            pltpu.make_async_copy(
                tab_hbm.at[pl.ds(0, ch)], bufs[b], sgs[b]).wait()

        def start_out(cc, b):
            off = pl.multiple_of(base + cc * ch, 8)
            pltpu.async_copy(bufs[b], out_hbm.at[pl.ds(off, ch)], sos[b])

        def wait_out(b):
            pltpu.make_async_copy(
                bufs[b], out_hbm.at[pl.ds(0, ch)], sos[b]).wait()

        for c in range(lag):
            start_gather(c, c % nbuf)

        def step(i, carry):
            for b in range(nbuf):
                c = i * nbuf + b
                bn = (b + lag) % nbuf

                @pl.when(c + lag < nchunk)
                def _():
                    @pl.when(c + lag >= nbuf)
                    def _():
                        wait_out(bn)
                    start_gather(c + lag, bn)

                wait_gather(b)
                start_out(c, b)
            return carry

        lax.fori_loop(0, nchunk // nbuf, step, 0)
        for b in range(nbuf):
            wait_out(b)

    return k(idx_all, table)


def _agg_math(neigh, self_v, wcol, pos, sess, w1h, w1w, w1p, w2r, w3a, w3b,
              m_blk):
    rows = m_blk * S
    f32 = jnp.float32
    e = neigh * sess
    t = jnp.dot(e, w1h, preferred_element_type=f32)
    t = t + jnp.dot(wcol, w1w, preferred_element_type=f32)
    t = t + jnp.dot(pos, w1p, preferred_element_type=f32)
    t = jnp.where(t >= 0, t, 0.2 * t)
    logit = jnp.sum(t * w2r, axis=1, keepdims=True)
    p = jnp.exp(logit - jnp.max(logit))
    gid = lax.broadcasted_iota(jnp.int32, (m_blk, rows), 0)
    rid = lax.broadcasted_iota(jnp.int32, (m_blk, rows), 1) // S
    grp = (gid == rid).astype(f32)
    num = jnp.dot(grp, p * neigh, preferred_element_type=f32)
    den = jnp.dot(grp, p, preferred_element_type=f32)
    agg = num / den
    out = jnp.dot(self_v, w3a, preferred_element_type=f32)
    out = out + jnp.dot(agg, w3b, preferred_element_type=f32)
    return jnp.maximum(out, 0.0)


_WSPECS = [
    pl.BlockSpec((D, D), lambda *a: (0, 0)),
    pl.BlockSpec((1, D), lambda *a: (0, 0)),
    pl.BlockSpec((P, D), lambda *a: (0, 0)),
    pl.BlockSpec((1, D), lambda *a: (0, 0)),
    pl.BlockSpec((D, D), lambda *a: (0, 0)),
    pl.BlockSpec((D, D), lambda *a: (0, 0)),
]


def _agg_a(h, g1i, mask3, wcol1, pos_1, wts):
    def body(h_ref, g1_ref, it_ref, mask_ref, wc_ref, pos_ref,
             w1h_ref, w1w_ref, w1p_ref, w2r_ref, w3a_ref, w3b_ref,
             out_ref, sess_ref):
        mask = mask_ref[0]
        items = it_ref[...]
        sess = jnp.dot(mask, items, preferred_element_type=jnp.float32)
        sess = sess / jnp.sum(mask)
        sess_ref[0] = sess
        out_ref[0] = _agg_math(
            g1_ref[...], h_ref[0], wc_ref[0], pos_ref[0], sess,
            w1h_ref[...], w1w_ref[...], w1p_ref[...], w2r_ref[...],
            w3a_ref[...], w3b_ref[...], L)

    return pl.pallas_call(
        body,
        grid=(B,),
        in_specs=[
            pl.BlockSpec((1, L, D), lambda b: (b, 0, 0)),
            pl.BlockSpec((M1, D), lambda b: (b, 0)),
            pl.BlockSpec((LP, D), lambda b: (N1 // LP + b, 0)),
            pl.BlockSpec((1, 1, LP), lambda b: (b, 0, 0)),
            pl.BlockSpec((1, M1, 1), lambda b: (b, 0, 0)),
            pl.BlockSpec((1, M1, P), lambda b: (b, 0, 0)),
        ] + _WSPECS,
        out_specs=[
            pl.BlockSpec((1, L, D), lambda b: (b, 0, 0)),
            pl.BlockSpec((1, 1, D), lambda b: (b, 0, 0)),
        ],
        out_shape=[
            jax.ShapeDtypeStruct((B, L, D), jnp.float32),
            jax.ShapeDtypeStruct((B, 1, D), jnp.float32),
        ],
    )(h, g1i, g1i, mask3, wcol1, pos_1, *wts)


MBLK_B = 40
NJ = M1 // MBLK_B


def _agg_b(kpart, g1i, g2k, wcol2, pos_2, sess, wts):
    b0 = kpart * BQ

    def body(self_ref, neigh_ref, wc_ref, pos_ref, sess_ref,
             w1h_ref, w1w_ref, w1p_ref, w2r_ref, w3a_ref, w3b_ref, out_ref):
        out_ref[0] = _agg_math(
            neigh_ref[...], self_ref[...], wc_ref[0], pos_ref[0], sess_ref[0],
            w1h_ref[...], w1w_ref[...], w1p_ref[...], w2r_ref[...],
            w3a_ref[...], w3b_ref[...], MBLK_B)

    return pl.pallas_call(
        body,
        grid=(BQ, NJ),
        in_specs=[
            pl.BlockSpec((MBLK_B, D),
                         lambda b, j: ((b0 + b) * NJ + j, 0)),
            pl.BlockSpec((MBLK_B * S, D), lambda b, j: (b * NJ + j, 0)),
            pl.BlockSpec((1, MBLK_B * S, 1), lambda b, j: (b0 + b, j, 0)),
            pl.BlockSpec((1, MBLK_B * S, P), lambda b, j: (b0 + b, j, 0)),
            pl.BlockSpec((1, 1, D), lambda b, j: (b0 + b, 0, 0)),
        ] + _WSPECS,
        out_specs=pl.BlockSpec((1, MBLK_B, D), lambda b, j: (b, j, 0)),
        out_shape=jax.ShapeDtypeStruct((BQ, M1, D), jnp.float32),
    )(g1i, g2k, wcol2, pos_2, sess, *wts)


def _agg_c(kpart, out1, out2k, wcol1, pos_1, sess, wts):
    b0 = kpart * BQ

    def body(self_ref, neigh_ref, wc_ref, pos_ref, sess_ref,
             w1h_ref, w1w_ref, w1p_ref, w2r_ref, w3a_ref, w3b_ref, out_ref):
        out_ref[0] = _agg_math(
            neigh_ref[0], self_ref[0], wc_ref[0], pos_ref[0], sess_ref[0],
            w1h_ref[...], w1w_ref[...], w1p_ref[...], w2r_ref[...],
            w3a_ref[...], w3b_ref[...], L)

    return pl.pallas_call(
        body,
        grid=(BQ,),
        in_specs=[
            pl.BlockSpec((1, L, D), lambda b: (b0 + b, 0, 0)),
            pl.BlockSpec((1, M1, D), lambda b: (b, 0, 0)),
            pl.BlockSpec((1, M1, 1), lambda b: (b0 + b, 0, 0)),
            pl.BlockSpec((1, M1, P), lambda b: (b0 + b, 0, 0)),
            pl.BlockSpec((1, 1, D), lambda b: (b0 + b, 0, 0)),
        ] + _WSPECS,
        out_specs=pl.BlockSpec((1, L, D), lambda b: (b, 0, 0)),
        out_shape=jax.ShapeDtypeStruct((BQ, L, D), jnp.float32),
    )(out1, out2k, wcol1, pos_1, sess, *wts)


def kernel(h, neighbors_1, neighbors_2, weights_1, weights_2, pos_1, pos_2,
           item, mask_item, embedding, w1_0, w2_0, w3_0, w1_1, w2_1, w3_1):
    item_p = jnp.concatenate(
        [item, jnp.zeros((B, LP - L), dtype=item.dtype)], axis=1)
    idx_all = jnp.concatenate([
        neighbors_1.reshape(-1), item_p.reshape(-1), neighbors_2.reshape(-1)
    ]).astype(jnp.int32)

    g1i = _gather_region(idx_all, embedding, 0, N1 + NI,
                         ch=120, nbuf=4, lag=2)
    g2 = [_gather_region(idx_all, embedding, N1 + NI + k * NPART, NPART,
                         ch=80, nbuf=5, lag=3)
          for k in range(KPART)]

    wcol1 = weights_1.reshape(B, M1, 1)
    wcol2 = weights_2.reshape(B, M2, 1)
    mask3 = jnp.concatenate(
        [mask_item, jnp.zeros((B, LP - L), dtype=mask_item.dtype)],
        axis=1).reshape(B, 1, LP)

    wts0 = (w1_0[:D], w1_0[D:D + 1], w1_0[D + 1:], w2_0.reshape(1, D),
            w3_0[:D], w3_0[D:])
    wts1 = (w1_1[:D], w1_1[D:D + 1], w1_1[D + 1:], w2_1.reshape(1, D),
            w3_1[:D], w3_1[D:])

    out1, sess = _agg_a(h, g1i, mask3, wcol1, pos_1, wts0)
    finals = []
    for k in range(KPART):
        out2k = _agg_b(k, g1i, g2[k], wcol2, pos_2, sess, wts0)
        finals.append(_agg_c(k, out1, out2k, wcol1, pos_1, sess, wts1))
    return jnp.concatenate(finals, axis=0)


# speedup vs baseline: 2.5024x; 1.8514x over previous
import functools

import jax
import jax.numpy as jnp
from jax import lax
from jax.experimental import pallas as pl
from jax.experimental.pallas import tpu as pltpu
from jax.experimental.pallas import tpu_sc as plsc

B, L, S, D, P = 128, 20, 10, 128, 16
M1 = L * S
M2 = L * S * S

LP = 2 * L
N1 = B * M1
NI = B * LP
N2 = B * M2

KPART = 4
BQ = B // KPART
NPART = BQ * M2

NC, NS = 2, 16
NW = NC * NS

GBA = 8
GBB = 2
GBC = 8


def _ring(tab_hbm, idx_v, out_hbm, base, npw, ch, bufs, sgs, sos, lag):
    nbuf = len(bufs)
    nchunk = npw // ch
    assert nchunk * ch == npw and nchunk % nbuf == 0 and 0 < lag < nbuf

    def start_gather(cc, b):
        ioff = pl.multiple_of(cc * ch, 8)
        pltpu.async_copy(
            tab_hbm.at[idx_v.at[pl.ds(ioff, ch)]], bufs[b], sgs[b])

    def wait_gather(b):
        pltpu.make_async_copy(
            tab_hbm.at[pl.ds(0, ch)], bufs[b], sgs[b]).wait()

    def start_out(cc, b):
        off = pl.multiple_of(base + cc * ch, 8)
        pltpu.async_copy(bufs[b], out_hbm.at[pl.ds(off, ch)], sos[b])

    def wait_out(b):
        pltpu.make_async_copy(
            bufs[b], out_hbm.at[pl.ds(0, ch)], sos[b]).wait()

    for c in range(lag):
        start_gather(c, c % nbuf)

    def step(i, carry):
        for b in range(nbuf):
            c = i * nbuf + b
            bn = (b + lag) % nbuf

            @pl.when(c + lag < nchunk)
            def _():
                @pl.when(c + lag >= nbuf)
                def _():
                    wait_out(bn)
                start_gather(c + lag, bn)

            wait_gather(b)
            start_out(c, b)
        return carry

    lax.fori_loop(0, nchunk // nbuf, step, 0)
    for b in range(nbuf):
        wait_out(b)


CH = 80
NBUF = 5


def _gather_quarter(idx2, table, kpart):
    mesh = plsc.VectorSubcoreMesh(core_axis_name="c", subcore_axis_name="s")
    npw = NPART // NW

    @functools.partial(
        pl.kernel,
        mesh=mesh,
        out_type=jax.ShapeDtypeStruct((NPART, D), jnp.float32),
        scratch_types=[
            pltpu.VMEM((npw,), jnp.int32),
        ] + [pltpu.VMEM((CH, D), jnp.float32)] * NBUF
          + [pltpu.SemaphoreType.DMA] * (2 * NBUF),
    )
    def k(idx_hbm, tab_hbm, out_hbm, idx_v, *rest):
        bufs = rest[:NBUF]
        sgs = rest[NBUF:2 * NBUF]
        sos = rest[2 * NBUF:]
        wid = lax.axis_index("s") * NC + lax.axis_index("c")
        base = wid * npw
        pltpu.sync_copy(idx_hbm.at[pl.ds(kpart * NPART + base, npw)], idx_v)
        _ring(tab_hbm, idx_v, out_hbm, base, npw, CH, bufs, sgs, sos, lag=3)

    return k(idx2, table)


def _gather_first(idx1, idxi, table):
    mesh = plsc.VectorSubcoreMesh(core_axis_name="c", subcore_axis_name="s")
    npw1 = N1 // NW
    npwi = NI // NW

    @functools.partial(
        pl.kernel,
        mesh=mesh,
        out_type=[
            jax.ShapeDtypeStruct((N1, D), jnp.float32),
            jax.ShapeDtypeStruct((NI, D), jnp.float32),
        ],
        scratch_types=[
            pltpu.VMEM((npw1,), jnp.int32),
            pltpu.VMEM((npwi,), jnp.int32),
        ] + [pltpu.VMEM((CH, D), jnp.float32)] * NBUF
          + [pltpu.SemaphoreType.DMA] * (2 * NBUF),
    )
    def k(idx1_hbm, idxi_hbm, tab_hbm, out1_hbm, outi_hbm,
          idx1_v, idxi_v, *rest):
        bufs = rest[:NBUF]
        sgs = rest[NBUF:2 * NBUF]
        sos = rest[2 * NBUF:]
        wid = lax.axis_index("s") * NC + lax.axis_index("c")
        pltpu.sync_copy(idx1_hbm.at[pl.ds(wid * npw1, npw1)], idx1_v)
        pltpu.sync_copy(idxi_hbm.at[pl.ds(wid * npwi, npwi)], idxi_v)
        _ring(tab_hbm, idx1_v, out1_hbm, wid * npw1, npw1, CH,
              bufs, sgs, sos, lag=3)
        _ring(tab_hbm, idxi_v, outi_hbm, wid * npwi, npwi, CH,
              bufs[:2], sgs[:2], sos[:2], lag=1)

    return k(idx1, idxi, table)


def _agg_math(neigh, self_v, wrows, pos, sess_blk, w1h, w1w, w1p, w2r,
              w3a, w3b, gb, m):
    rows = gb * m * S
    groups = gb * m
    f32 = jnp.float32
    bid = lax.broadcasted_iota(jnp.int32, (rows, gb), 0) // (m * S)
    cid = lax.broadcasted_iota(jnp.int32, (rows, gb), 1)
    ab = (bid == cid).astype(f32)
    sess_rows = jnp.dot(ab, sess_blk, preferred_element_type=f32)
    e = neigh * sess_rows
    t = jnp.dot(e, w1h, preferred_element_type=f32)
    t = t + jnp.dot(pos, w1p, preferred_element_type=f32)
    wparts = [
        lax.dot_general(wrows[g], w1w, (((0,), (0,)), ((), ())),
                        preferred_element_type=f32)
        for g in range(gb)
    ]
    t = t + (jnp.concatenate(wparts, axis=0) if gb > 1 else wparts[0])
    t = jnp.where(t >= 0, t, 0.2 * t)
    logit = jnp.sum(t * w2r, axis=1, keepdims=True)
    p = jnp.exp(logit - jnp.max(logit))
    pn = (p * neigh).reshape(groups, S, D)
    num = jnp.sum(pn, axis=1)
    gid2 = lax.broadcasted_iota(jnp.int32, (groups, rows), 0)
    rid2 = lax.broadcasted_iota(jnp.int32, (groups, rows), 1) // S
    grp = (gid2 == rid2).astype(f32)
    den = jnp.dot(grp, p, preferred_element_type=f32)
    agg = num / den
    out = jnp.dot(self_v, w3a, preferred_element_type=f32)
    out = out + jnp.dot(agg, w3b, preferred_element_type=f32)
    return jnp.maximum(out, 0.0)


_WSPECS = [
    pl.BlockSpec((D, D), lambda *a: (0, 0)),
    pl.BlockSpec((1, D), lambda *a: (0, 0)),
    pl.BlockSpec((P, D), lambda *a: (0, 0)),
    pl.BlockSpec((1, D), lambda *a: (0, 0)),
    pl.BlockSpec((D, D), lambda *a: (0, 0)),
    pl.BlockSpec((D, D), lambda *a: (0, 0)),
]


def _agg_a(h, g1, items3, maskr, wrow1, pos_1, wts):
    def body(h_ref, g1_ref, it_ref, mask_ref, wr_ref, pos_ref,
             w1h_ref, w1w_ref, w1p_ref, w2r_ref, w3a_ref, w3b_ref,
             out_ref, sess_ref):
        f32 = jnp.float32
        items = it_ref[...].reshape(GBA * LP, D)
        mask = mask_ref[0]
        gid = lax.broadcasted_iota(jnp.int32, (GBA, GBA * LP), 0)
        rid = lax.broadcasted_iota(jnp.int32, (GBA, GBA * LP), 1) // LP
        mmat = jnp.where(gid == rid, mask, 0.0)
        sess_blk = jnp.dot(mmat, items, preferred_element_type=f32)
        sess_blk = sess_blk / jnp.sum(mmat, axis=1, keepdims=True)
        sess_ref[...] = sess_blk.reshape(GBA, 1, D)
        out = _agg_math(
            g1_ref[...].reshape(GBA * M1, D),
            h_ref[...].reshape(GBA * L, D),
            wr_ref[...], pos_ref[...].reshape(GBA * M1, P), sess_blk,
            w1h_ref[...], w1w_ref[...], w1p_ref[...], w2r_ref[...],
            w3a_ref[...], w3b_ref[...], GBA, L)
        out_ref[...] = out.reshape(GBA, L, D)

    return pl.pallas_call(
        body,
        grid=(B // GBA,),
        in_specs=[
            pl.BlockSpec((GBA, L, D), lambda g: (g, 0, 0)),
            pl.BlockSpec((GBA, M1, D), lambda g: (g, 0, 0)),
            pl.BlockSpec((GBA, LP, D), lambda g: (g, 0, 0)),
            pl.BlockSpec((1, 1, GBA * LP), lambda g: (g, 0, 0)),
            pl.BlockSpec((GBA, 1, M1), lambda g: (g, 0, 0)),
            pl.BlockSpec((GBA, M1, P), lambda g: (g, 0, 0)),
        ] + _WSPECS,
        out_specs=[
            pl.BlockSpec((GBA, L, D), lambda g: (g, 0, 0)),
            pl.BlockSpec((GBA, 1, D), lambda g: (g, 0, 0)),
        ],
        out_shape=[
            jax.ShapeDtypeStruct((B, L, D), jnp.float32),
            jax.ShapeDtypeStruct((B, 1, D), jnp.float32),
        ],
    )(h, g1, items3, maskr, wrow1, pos_1, *wts)


def _agg_b(kpart, g1, g2k, wrow2, pos_2, sess, wts):
    k16 = kpart * (BQ // GBB)

    def body(self_ref, neigh_ref, wr_ref, pos_ref, sess_ref,
             w1h_ref, w1w_ref, w1p_ref, w2r_ref, w3a_ref, w3b_ref, out_ref):
        out = _agg_math(
            neigh_ref[...].reshape(GBB * M2, D),
            self_ref[...].reshape(GBB * M1, D),
            wr_ref[...], pos_ref[...].reshape(GBB * M2, P),
            sess_ref[...].reshape(GBB, D),
            w1h_ref[...], w1w_ref[...], w1p_ref[...], w2r_ref[...],
            w3a_ref[...], w3b_ref[...], GBB, M1)
        out_ref[...] = out.reshape(GBB, M1, D)

    return pl.pallas_call(
        body,
        grid=(BQ // GBB,),
        in_specs=[
            pl.BlockSpec((GBB, M1, D), lambda g: (k16 + g, 0, 0)),
            pl.BlockSpec((GBB, M2, D), lambda g: (g, 0, 0)),
            pl.BlockSpec((GBB, 1, M2), lambda g: (k16 + g, 0, 0)),
            pl.BlockSpec((GBB, M2, P), lambda g: (k16 + g, 0, 0)),
            pl.BlockSpec((GBB, 1, D), lambda g: (k16 + g, 0, 0)),
        ] + _WSPECS,
        out_specs=pl.BlockSpec((GBB, M1, D), lambda g: (g, 0, 0)),
        out_shape=jax.ShapeDtypeStruct((BQ, M1, D), jnp.float32),
    )(g1, g2k, wrow2, pos_2, sess, *wts)


def _agg_c(kpart, out1, out2k, wrow1, pos_1, sess, wts):
    k4 = kpart * (BQ // GBC)

    def body(self_ref, neigh_ref, wr_ref, pos_ref, sess_ref,
             w1h_ref, w1w_ref, w1p_ref, w2r_ref, w3a_ref, w3b_ref, out_ref):
        out = _agg_math(
            neigh_ref[...].reshape(GBC * M1, D),
            self_ref[...].reshape(GBC * L, D),
            wr_ref[...], pos_ref[...].reshape(GBC * M1, P),
            sess_ref[...].reshape(GBC, D),
            w1h_ref[...], w1w_ref[...], w1p_ref[...], w2r_ref[...],
            w3a_ref[...], w3b_ref[...], GBC, L)
        out_ref[...] = out.reshape(GBC, L, D)

    return pl.pallas_call(
        body,
        grid=(BQ // GBC,),
        in_specs=[
            pl.BlockSpec((GBC, L, D), lambda g: (k4 + g, 0, 0)),
            pl.BlockSpec((GBC, M1, D), lambda g: (g, 0, 0)),
            pl.BlockSpec((GBC, 1, M1), lambda g: (k4 + g, 0, 0)),
            pl.BlockSpec((GBC, M1, P), lambda g: (k4 + g, 0, 0)),
            pl.BlockSpec((GBC, 1, D), lambda g: (k4 + g, 0, 0)),
        ] + _WSPECS,
        out_specs=pl.BlockSpec((GBC, L, D), lambda g: (g, 0, 0)),
        out_shape=jax.ShapeDtypeStruct((BQ, L, D), jnp.float32),
    )(out1, out2k, wrow1, pos_1, sess, *wts)


def kernel(h, neighbors_1, neighbors_2, weights_1, weights_2, pos_1, pos_2,
           item, mask_item, embedding, w1_0, w2_0, w3_0, w1_1, w2_1, w3_1):
    item_p = jnp.concatenate(
        [item, jnp.zeros((B, LP - L), dtype=item.dtype)], axis=1)
    idx1 = neighbors_1.reshape(-1).astype(jnp.int32)
    idxi = item_p.reshape(-1).astype(jnp.int32)
    idx2 = neighbors_2.reshape(-1).astype(jnp.int32)

    g1_rows, item_rows = _gather_first(idx1, idxi, embedding)
    g2 = [_gather_quarter(idx2, embedding, k).reshape(BQ, M2, D)
          for k in range(KPART)]
    g1 = g1_rows.reshape(B, M1, D)
    items3 = item_rows.reshape(B, LP, D)

    wrow1 = weights_1.reshape(B, 1, M1)
    wrow2 = weights_2.reshape(B, 1, M2)
    maskr = jnp.concatenate(
        [mask_item, jnp.zeros((B, LP - L), dtype=mask_item.dtype)],
        axis=1).reshape(B // GBA, 1, GBA * LP)

    wts0 = (w1_0[:D], w1_0[D:D + 1], w1_0[D + 1:], w2_0.reshape(1, D),
            w3_0[:D], w3_0[D:])
    wts1 = (w1_1[:D], w1_1[D:D + 1], w1_1[D + 1:], w2_1.reshape(1, D),
            w3_1[:D], w3_1[D:])

    out1, sess = _agg_a(h, g1, items3, maskr, wrow1, pos_1, wts0)
    finals = []
    for k in range(KPART):
        out2k = _agg_b(k, g1, g2[k], wrow2, pos_2, sess, wts0)
        finals.append(_agg_c(k, out1, out2k, wrow1, pos_1, sess, wts1))
    return jnp.concatenate(finals, axis=0)
